# baseline (speedup 1.0000x reference)
import jax
import jax.numpy as jnp
from jax import lax
from jax.experimental import pallas as pl
from jax.experimental.pallas import tpu as pltpu
from jax.experimental.pallas import tpu_sc as plsc

B, S, D = 4, 8192, 768
NC, NS = 2, 16
NW = NC * NS
S_PER_W = S // NW
CHUNK = 16
STEPS = S_PER_W // CHUNK
CW = CHUNK * D
LANES = 16
UNROLL = 8
K = STEPS
NXB = 6
LOOKAHEAD = 3
BS = 512


def _sc_body(x_hbm, pos_hbm, out_hbm, *refs):
    xb = refs[:NXB]
    pb = refs[NXB:NXB + 2]
    xin = refs[NXB + 2:2 * NXB + 2]
    xout = refs[2 * NXB + 2:3 * NXB + 2]
    ps = refs[3 * NXB + 2:3 * NXB + 4]

    wid = lax.axis_index("s") * NC + lax.axis_index("c")
    base = wid * S_PER_W * D

    def off(k):
        return base + k * CW

    pending_in = {}
    pending_out = {}
    pending_p = {}

    def start_p(t):
        pending_p[t] = pltpu.async_copy(
            pos_hbm.at[pl.ds(off(t), CW)], pb[t % 2], ps[t % 2])

    def start_in(k):
        pending_in[k] = pltpu.async_copy(
            x_hbm.at[pl.ds(off(k), CW)], xb[k % NXB], xin[k % NXB])

    def start_out(k):
        pending_out[k] = pltpu.async_copy(
            xb[k % NXB], out_hbm.at[pl.ds(off(k), CW)], xout[k % NXB])

    start_p(0)
    for k in range(LOOKAHEAD):
        start_in(k)

    for k in range(K):
        pending_p.pop(k).wait()
        if k + 1 < STEPS:
            start_p(k + 1)
        pending_in.pop(k).wait()

        xv, pv = xb[k % NXB], pb[k % 2]

        @plsc.parallel_loop(0, CW // LANES, 1, unroll=UNROLL)
        def add_body(i, xv=xv, pv=pv):
            o = i * LANES
            xv[pl.ds(o, LANES)] = xv[pl.ds(o, LANES)] + pv[pl.ds(o, LANES)]

        start_out(k)
        if k + LOOKAHEAD < K:
            prev = k + LOOKAHEAD - NXB
            if prev >= 0:
                pending_out.pop(prev).wait()
            start_in(k + LOOKAHEAD)

    for k in sorted(pending_out):
        pending_out.pop(k).wait()


def _tc_body(x_ref, p_ref, o_ref):
    o_ref[...] = x_ref[...] + p_ref[...][None]


@jax.jit
def _pos_add(x, pos):
    x0_flat = x[0].reshape(-1)
    pos_flat = pos.reshape(-1)
    mesh = plsc.VectorSubcoreMesh(core_axis_name="c", subcore_axis_name="s")
    sc_out = pl.kernel(
        _sc_body,
        mesh=mesh,
        out_type=jax.ShapeDtypeStruct((S * D,), jnp.float32),
        scratch_types=(
            [pltpu.VMEM((CW,), jnp.float32)] * (NXB + 2)
            + [pltpu.SemaphoreType.DMA] * (2 * NXB + 2)
        ),
    )(x0_flat, pos_flat)

    tc_out = pl.pallas_call(
        _tc_body,
        grid=(S // BS, B - 1),
        in_specs=[
            pl.BlockSpec((1, BS, D), lambda i, b: (b, i, 0)),
            pl.BlockSpec((BS, D), lambda i, b: (i, 0)),
        ],
        out_specs=pl.BlockSpec((1, BS, D), lambda i, b: (b, i, 0)),
        out_shape=jax.ShapeDtypeStruct((B - 1, S, D), jnp.float32),
    )(x[1:], pos)

    return jnp.concatenate([sc_out.reshape(1, S, D), tc_out], axis=0)


def kernel(x, pos_table):
    return _pos_add(x, pos_table)


# speedup vs baseline: 2.7008x; 2.7008x over previous
import jax
import jax.numpy as jnp
from jax import lax
from jax.experimental import pallas as pl
from jax.experimental.pallas import tpu as pltpu
from jax.experimental.pallas import tpu_sc as plsc

B, S, D = 4, 8192, 768
NC, NS = 2, 16
NW = NC * NS
S_PER_W = S // NW
CHUNK = 16
STEPS = S_PER_W // CHUNK
LANES = 16
SLICES = D // LANES


def _body(x_hbm, pos_hbm, out_hbm, *refs):
    xb = refs[:B]
    pb = refs[B:B + 2]
    xin = refs[B + 2:2 * B + 2]
    xout = refs[2 * B + 2:3 * B + 2]
    ps = refs[3 * B + 2:3 * B + 4]

    wid = lax.axis_index("s") * NC + lax.axis_index("c")
    s_base = wid * S_PER_W

    def rows(t):
        return pl.ds(s_base + t * CHUNK, CHUNK)

    def in_copy(t, b):
        return pltpu.make_async_copy(x_hbm.at[b, rows(t)], xb[b], xin[b])

    def out_copy(t, b):
        return pltpu.make_async_copy(xb[b], out_hbm.at[b, rows(t)], xout[b])

    def p_copy(t, jp):
        return pltpu.make_async_copy(pos_hbm.at[rows(t)], pb[jp], ps[jp])

    def chunk_work(i, t, parity):
        pv = pb[parity]
        p_copy(jnp.minimum(t + 1, STEPS - 1), 1 - parity).start()
        p_copy(t, parity).wait()
        for b in range(B):
            in_copy(t, b).wait()
            xv = xb[b]

            @plsc.parallel_loop(0, CHUNK, 1)
            def add_body(r, xv=xv, pv=pv):
                for u in range(SLICES):
                    o = u * LANES
                    xv[r, pl.ds(o, LANES)] = (
                        xv[r, pl.ds(o, LANES)] + pv[r, pl.ds(o, LANES)])

            out_copy(t, b).start()
            if b < 2:
                if parity == 0:
                    @pl.when(i > 0)
                    def _():
                        out_copy(t, b + 2).wait()
                else:
                    out_copy(t, b + 2).wait()
                in_copy(t, b + 2).start()
            else:
                if parity == 1:
                    @pl.when(i < STEPS // 2 - 1)
                    def _():
                        out_copy(t, b - 2).wait()
                        in_copy(t + 1, b - 2).start()
                else:
                    out_copy(t, b - 2).wait()
                    in_copy(t + 1, b - 2).start()

    p_copy(0, 0).start()
    in_copy(0, 0).start()
    in_copy(0, 1).start()

    def t_pair(i, _):
        t = 2 * i
        chunk_work(i, t, 0)
        chunk_work(i, t + 1, 1)
        return 0

    lax.fori_loop(0, STEPS // 2, t_pair, 0)

    p_copy(STEPS - 1, STEPS % 2).wait()
    for b in range(B):
        out_copy(STEPS - 1, b).wait()


@jax.jit
def _pos_add(x, pos):
    mesh = plsc.VectorSubcoreMesh(core_axis_name="c", subcore_axis_name="s")
    return pl.kernel(
        _body,
        mesh=mesh,
        out_type=jax.ShapeDtypeStruct((B, S, D), jnp.float32),
        scratch_types=(
            [pltpu.VMEM((CHUNK, D), jnp.float32)] * (B + 2)
            + [pltpu.SemaphoreType.DMA] * (2 * B + 2)
        ),
        compiler_params=pltpu.CompilerParams(use_tc_tiling_on_sc=True),
    )(x, pos)


def kernel(x, pos_table):
    return _pos_add(x, pos_table)


# speedup vs baseline: 3.0015x; 1.1113x over previous
import jax
import jax.numpy as jnp
from jax import lax
from jax.experimental import pallas as pl
from jax.experimental.pallas import tpu as pltpu
from jax.experimental.pallas import tpu_sc as plsc

B, S, D = 4, 8192, 768
NC, NS = 2, 16
NW = NC * NS
S_PER_W = S // NW
CHUNK = 16
STEPS = S_PER_W // CHUNK
LANES = 16
SLICES = D // LANES
NXB = 2 * B


def _body(x_hbm, pos_hbm, out_hbm, *refs):
    xb = refs[:NXB]
    pb = refs[NXB:NXB + 2]
    xin = refs[NXB + 2:2 * NXB + 2]
    xout = refs[2 * NXB + 2:3 * NXB + 2]
    ps = refs[3 * NXB + 2:3 * NXB + 4]

    wid = lax.axis_index("s") * NC + lax.axis_index("c")
    s_base = wid * S_PER_W

    def rows(t):
        return pl.ds(s_base + t * CHUNK, CHUNK)

    def in_copy(t, b, j):
        return pltpu.make_async_copy(x_hbm.at[b, rows(t)], xb[j], xin[j])

    def out_copy(t, b, j):
        return pltpu.make_async_copy(xb[j], out_hbm.at[b, rows(t)], xout[j])

    def p_copy(t, jp):
        return pltpu.make_async_copy(pos_hbm.at[rows(t)], pb[jp], ps[jp])

    def chunk_work(i, t, parity):
        q = parity * B
        oq = (1 - parity) * B
        pv = pb[parity]
        p_copy(jnp.minimum(t + 1, STEPS - 1), 1 - parity).start()
        p_copy(t, parity).wait()
        for b in range(B):
            in_copy(t, b, q + b).wait()
            xv = xb[q + b]

            @plsc.parallel_loop(0, CHUNK, 1)
            def add_body(r, xv=xv, pv=pv):
                for u in range(SLICES):
                    o = u * LANES
                    xv[r, pl.ds(o, LANES)] = (
                        xv[r, pl.ds(o, LANES)] + pv[r, pl.ds(o, LANES)])

            out_copy(t, b, q + b).start()
            if parity == 0:
                @pl.when(i > 0)
                def _():
                    out_copy(t, b, oq + b).wait()
                in_copy(t + 1, b, oq + b).start()
            else:
                out_copy(t, b, oq + b).wait()

                @pl.when(i < STEPS // 2 - 1)
                def _():
                    in_copy(t + 1, b, oq + b).start()

    p_copy(0, 0).start()
    for b in range(B):
        in_copy(0, b, b).start()

    def t_pair(i, _):
        chunk_work(i, 2 * i, 0)
        chunk_work(i, 2 * i + 1, 1)
        return 0

    lax.fori_loop(0, STEPS // 2, t_pair, 0)

    p_copy(STEPS - 1, STEPS % 2).wait()
    for b in range(B):
        out_copy(STEPS - 1, b, B + b).wait()


@jax.jit
def _pos_add(x, pos):
    mesh = plsc.VectorSubcoreMesh(core_axis_name="c", subcore_axis_name="s")
    return pl.kernel(
        _body,
        mesh=mesh,
        out_type=jax.ShapeDtypeStruct((B, S, D), jnp.float32),
        scratch_types=(
            [pltpu.VMEM((CHUNK, D), jnp.float32)] * (NXB + 2)
            + [pltpu.SemaphoreType.DMA] * (2 * NXB + 2)
        ),
        compiler_params=pltpu.CompilerParams(use_tc_tiling_on_sc=True),
    )(x, pos)


def kernel(x, pos_table):
    return _pos_add(x, pos_table)
